# baseline (speedup 1.0000x reference)
import jax
import jax.numpy as jnp
from jax.experimental import pallas as pl
from jax.experimental.pallas import tpu as pltpu

_NUM_CONTEXTS = 32
_NUM_GENS = 96
_MAX_DRAFT = 3
_ROWS = _NUM_CONTEXTS + _NUM_GENS * (_MAX_DRAFT + 1)
_VOCAB = 100000
_RB = 8
_NB = _ROWS // _RB


def _argmax_body(x_ref, draft_ref, tt_ref, val_ref, acc_ref, a_scr):
    i = pl.program_id(0)
    x = x_ref[...]
    col = jax.lax.broadcasted_iota(jnp.int32, (_RB, _VOCAB), 1)
    lmax = jnp.max(x, axis=1, keepdims=True)
    larg = jnp.min(jnp.where(x == lmax, col, _VOCAB), axis=1, keepdims=True)
    tt_ref[...] = larg
    val_ref[...] = lmax
    a_scr[pl.ds(i * _RB, _RB), :] = larg

    @pl.when(i == _NB - 1)
    def _fin():
        tt = a_scr[...]
        gen_t = tt[_NUM_CONTEXTS:, 0].reshape(_NUM_GENS, _MAX_DRAFT + 1)
        draft = draft_ref[...]
        m = (draft == gen_t[:, :_MAX_DRAFT]).astype(jnp.int32)
        run = m[:, 0:1]
        total = run
        for k in range(1, _MAX_DRAFT):
            run = run * m[:, k:k + 1]
            total = total + run
        acc_ref[...] = 1 + total


@jax.jit
def kernel(logits, draft_tokens):
    if logits.ndim == 1:
        logits = logits[None, :]
    draft_tokens = draft_tokens.astype(jnp.int32)
    tt, vals, num_acc_gen = pl.pallas_call(
        _argmax_body,
        grid=(_NB,),
        in_specs=[
            pl.BlockSpec((_RB, _VOCAB), lambda i: (i, 0)),
            pl.BlockSpec((_NUM_GENS, _MAX_DRAFT), lambda i: (0, 0)),
        ],
        out_specs=[
            pl.BlockSpec((_RB, 1), lambda i: (i, 0)),
            pl.BlockSpec((_RB, 1), lambda i: (i, 0)),
            pl.BlockSpec((_NUM_GENS, 1), lambda i: (0, 0)),
        ],
        out_shape=[
            jax.ShapeDtypeStruct((_ROWS, 1), jnp.int32),
            jax.ShapeDtypeStruct((_ROWS, 1), jnp.float32),
            jax.ShapeDtypeStruct((_NUM_GENS, 1), jnp.int32),
        ],
        scratch_shapes=[
            pltpu.VMEM((_ROWS, 1), jnp.int32),
        ],
    )(logits, draft_tokens)

    target_tokens = tt[:, 0]
    accepted_values = vals[:, 0]
    ctx_accepted = jnp.concatenate(
        [target_tokens[:_NUM_CONTEXTS, None],
         jnp.zeros((_NUM_CONTEXTS, _MAX_DRAFT), dtype=jnp.int32)], axis=1)
    gen_accepted = target_tokens[_NUM_CONTEXTS:].reshape(_NUM_GENS, _MAX_DRAFT + 1)
    accepted_tokens = jnp.concatenate([ctx_accepted, gen_accepted], axis=0)
    num_accepted = jnp.concatenate(
        [jnp.ones((_NUM_CONTEXTS,), dtype=jnp.int32), num_acc_gen[:, 0]], axis=0)
    return accepted_tokens, num_accepted, accepted_values
